# baseline (speedup 1.0000x reference)
import functools

import jax
import jax.numpy as jnp
from jax import lax
from jax.experimental import pallas as pl
from jax.experimental.pallas import tpu as pltpu
from jax.experimental.pallas import tpu_sc as plsc

N_E = 1024
E_DIM = 64
BETA = 0.25
TN = 1024


def _vq_body(z_ref, wt_ref, *rest, n_total, total_steps, local_steps,
             has_carry):
    if has_carry:
        (_ohbuf, _idxbuf, loss_in, counts_in, oh_ref, idx_ref, idxsep_ref,
         loss_ref, counts_ref, perp_ref) = rest
    else:
        (oh_ref, idx_ref, idxsep_ref, loss_ref, counts_ref,
         perp_ref) = rest
    step = pl.program_id(0)

    z = z_ref[...]
    z2 = z + z
    wt = wt_ref[...]

    dot2 = jax.lax.dot_general(z2, wt, (((1,), (0,)), ((), ())),
                               preferred_element_type=jnp.float32)
    z_sq = 0.25 * jnp.sum(z2 * z2, axis=1, keepdims=True)
    e_sq = jnp.sum(wt * wt, axis=0, keepdims=True)
    d = (z_sq + e_sq) - dot2

    d_min = jnp.min(d, axis=1, keepdims=True)
    fiota = jax.lax.broadcasted_iota(jnp.int32, (TN, N_E), 1).astype(jnp.float32)
    idx_f = jnp.min(jnp.where(d == d_min, fiota, float(N_E)),
                    axis=1, keepdims=True)
    idx_row = jnp.transpose(idx_f.astype(jnp.int32))
    idx_ref[0] = idx_row
    idxsep_ref[0] = idx_row

    one_hot = (fiota == idx_f).astype(jnp.float32)
    oh_ref[...] = one_hot

    @pl.when(step == 0)
    def _init():
        if has_carry:
            loss_ref[...] = loss_in[...]
            counts_ref[...] = counts_in[...]
        else:
            loss_ref[...] = jnp.zeros_like(loss_ref)
            counts_ref[...] = jnp.zeros_like(counts_ref)
        perp_ref[...] = jnp.zeros_like(perp_ref)

    loss_ref[...] += jnp.full(loss_ref.shape, jnp.sum(d_min), jnp.float32)
    ones_row = jnp.ones((1, TN), jnp.float32)
    counts_ref[...] += jax.lax.dot_general(
        ones_row, one_hot, (((1,), (0,)), ((), ())),
        preferred_element_type=jnp.float32)

    if has_carry:
        @pl.when(step == local_steps - 1)
        def _finalize():
            loss_ref[...] = loss_ref[...] * (BETA / (n_total * E_DIM))
            p = counts_ref[...] / n_total
            ent = -jnp.sum(p * jnp.log(p + 1e-10))
            perp_ref[...] = jnp.full(perp_ref.shape, jnp.exp(ent),
                                     jnp.float32)


def _tc_call(z_flat, wt, off, local_steps, total_steps, n, carry=None):
    has_carry = carry is not None
    out_shapes = (
        jax.ShapeDtypeStruct((n, N_E), jnp.float32),
        jax.ShapeDtypeStruct((total_steps, 1, TN), jnp.int32),
        jax.ShapeDtypeStruct((local_steps, 1, TN), jnp.int32),
        jax.ShapeDtypeStruct((1, 128), jnp.float32),
        jax.ShapeDtypeStruct((1, N_E), jnp.float32),
        jax.ShapeDtypeStruct((1, 128), jnp.float32),
    )
    in_specs = [
        pl.BlockSpec((TN, E_DIM), lambda i, o=off: (i + o, 0)),
        pl.BlockSpec((E_DIM, N_E), lambda i: (0, 0)),
    ]
    operands = [z_flat, wt]
    aliases = {}
    if has_carry:
        oh_buf, idx_buf, loss_in, counts_in = carry
        in_specs += [
            pl.BlockSpec(memory_space=pl.ANY),
            pl.BlockSpec(memory_space=pl.ANY),
            pl.BlockSpec((1, 128), lambda i: (0, 0)),
            pl.BlockSpec((1, N_E), lambda i: (0, 0)),
        ]
        operands += [oh_buf, idx_buf, loss_in, counts_in]
        aliases = {2: 0, 3: 1}
    out_specs = (
        pl.BlockSpec((TN, N_E), lambda i, o=off: (i + o, 0)),
        pl.BlockSpec((1, 1, TN), lambda i, o=off: (i + o, 0, 0)),
        pl.BlockSpec((1, 1, TN), lambda i: (i, 0, 0)),
        pl.BlockSpec((1, 128), lambda i: (0, 0)),
        pl.BlockSpec((1, N_E), lambda i: (0, 0)),
        pl.BlockSpec((1, 128), lambda i: (0, 0)),
    )
    return pl.pallas_call(
        functools.partial(_vq_body, n_total=n, total_steps=total_steps,
                          local_steps=local_steps, has_carry=has_carry),
        grid=(local_steps,),
        in_specs=in_specs,
        out_specs=out_specs,
        out_shape=out_shapes,
        input_output_aliases=aliases,
        compiler_params=pltpu.CompilerParams(
            dimension_semantics=("arbitrary",)),
    )(*operands)


def _make_sc_gather(n_half):
    info = plsc.get_sparse_core_info()
    nc, ns = info.num_cores, info.num_subcores
    nw = nc * ns
    b_per_w = n_half // nw
    chunk = 128
    quarter = b_per_w // 4
    n_q = b_per_w // quarter
    mesh = plsc.VectorSubcoreMesh(core_axis_name="c", subcore_axis_name="s")

    @functools.partial(
        pl.kernel, mesh=mesh,
        out_type=jax.ShapeDtypeStruct((n_half, 2 * E_DIM), jnp.float32),
        scratch_types=[
            pltpu.VMEM((b_per_w,), jnp.int32),
            pltpu.VMEM((2, quarter, 2 * E_DIM), jnp.float32),
            pltpu.SemaphoreType.DMA,
            pltpu.SemaphoreType.DMA,
        ],
    )
    def gather(table_hbm, idx_hbm, out_hbm, idx_v, rows_v, semg, semo):
        wid = lax.axis_index("s") * nc + lax.axis_index("c")
        base = wid * b_per_w
        pltpu.sync_copy(idx_hbm.at[pl.ds(base, b_per_w)], idx_v)

        def fire(q, buf):
            return [pltpu.async_copy(
                table_hbm.at[idx_v.at[pl.ds(q * quarter + j * chunk, chunk)]],
                rows_v.at[buf, pl.ds(j * chunk, chunk)], semg)
                for j in range(quarter // chunk)]

        gh = fire(0, 0)
        oh = []
        for q in range(n_q):
            for c in gh:
                c.wait()
            oh.append(pltpu.async_copy(
                rows_v.at[q % 2],
                out_hbm.at[pl.ds(base + q * quarter, quarter)], semo))
            if q + 1 < n_q:
                if q >= 1:
                    oh[q - 1].wait()
                gh = fire(q + 1, (q + 1) % 2)
        oh[n_q - 2].wait()
        oh[n_q - 1].wait()

    return gather


def kernel(z, W):
    B, C, H, Wd = z.shape
    n = B * H * Wd
    total_steps = n // TN
    half_steps = total_steps // 2
    n_half = n // 2
    z_flat = jnp.transpose(z, (0, 2, 3, 1)).reshape(n, E_DIM)
    wt = W.T
    w_pad = jnp.pad(W, ((0, 0), (0, E_DIM)))

    oh1, idxf1, idxa, loss1, counts1, _ = _tc_call(
        z_flat, wt, 0, half_steps, total_steps, n)
    sc_gather = _make_sc_gather(n_half)
    zq_a = sc_gather(w_pad, idxa.reshape(n_half))

    oh2, idxf2, idxb, loss_o, _counts, perp_o = _tc_call(
        z_flat, wt, half_steps, half_steps, total_steps, n,
        carry=(oh1, idxf1, loss1, counts1))
    zq_b = sc_gather(w_pad, idxb.reshape(n_half))

    one_hot = oh2
    indices = idxf2.reshape(n)
    zq_flat = jnp.concatenate([zq_a[:, :E_DIM], zq_b[:, :E_DIM]], axis=0)
    z_q = jnp.transpose(zq_flat.reshape(B, H, Wd, E_DIM), (0, 3, 1, 2))
    loss = loss_o[0, 0]
    perplexity = perp_o[0, 0]
    return (loss, z_q, perplexity, one_hot, indices)


# speedup vs baseline: 1.1978x; 1.1978x over previous
import functools

import jax
import jax.numpy as jnp
from jax import lax
from jax.experimental import pallas as pl
from jax.experimental.pallas import tpu as pltpu
from jax.experimental.pallas import tpu_sc as plsc

N_E = 1024
E_DIM = 64
BETA = 0.25
TN = 1024


def _vq_kernel(z_ref, wt_ref, oh_ref, idx_ref, loss_ref,
               counts_ref, perp_ref, *, n_total, n_steps):
    step = pl.program_id(0)

    z = z_ref[...]
    z2 = z + z
    wt = wt_ref[...]

    dot2 = jax.lax.dot_general(z2, wt, (((1,), (0,)), ((), ())),
                               preferred_element_type=jnp.float32)
    z_sq = 0.25 * jnp.sum(z2 * z2, axis=1, keepdims=True)
    e_sq = jnp.sum(wt * wt, axis=0, keepdims=True)
    d = (z_sq + e_sq) - dot2

    d_min = jnp.min(d, axis=1, keepdims=True)
    fiota = jax.lax.broadcasted_iota(jnp.int32, (TN, N_E), 1).astype(jnp.float32)
    idx_f = jnp.min(jnp.where(d == d_min, fiota, float(N_E)),
                    axis=1, keepdims=True)
    idx_ref[0] = jnp.transpose(idx_f.astype(jnp.int32))

    one_hot = (fiota == idx_f).astype(jnp.float32)
    oh_ref[...] = one_hot

    @pl.when(step == 0)
    def _init():
        loss_ref[...] = jnp.zeros_like(loss_ref)
        counts_ref[...] = jnp.zeros_like(counts_ref)
        perp_ref[...] = jnp.zeros_like(perp_ref)

    loss_ref[...] += jnp.full(loss_ref.shape, jnp.sum(d_min), jnp.float32)
    ones_row = jnp.ones((1, TN), jnp.float32)
    counts_ref[...] += jax.lax.dot_general(
        ones_row, one_hot, (((1,), (0,)), ((), ())),
        preferred_element_type=jnp.float32)

    @pl.when(step == n_steps - 1)
    def _finalize():
        loss_ref[...] = loss_ref[...] * (BETA / (n_total * E_DIM))
        p = counts_ref[...] / n_total
        ent = -jnp.sum(p * jnp.log(p + 1e-10))
        perp_ref[...] = jnp.full(perp_ref.shape, jnp.exp(ent), jnp.float32)


def _make_sc_gather(n):
    info = plsc.get_sparse_core_info()
    nc, ns = info.num_cores, info.num_subcores
    nw = nc * ns
    b_per_w = n // nw
    half = b_per_w // 2
    chunk = 128
    mesh = plsc.VectorSubcoreMesh(core_axis_name="c", subcore_axis_name="s")

    quarter = b_per_w // 4
    n_q = b_per_w // quarter

    @functools.partial(
        pl.kernel, mesh=mesh,
        out_type=jax.ShapeDtypeStruct((n, 2 * E_DIM), jnp.float32),
        scratch_types=[
            pltpu.VMEM((b_per_w,), jnp.int32),
            pltpu.VMEM((2, quarter, 2 * E_DIM), jnp.float32),
            pltpu.SemaphoreType.DMA,
            pltpu.SemaphoreType.DMA,
        ],
    )
    def gather(table_hbm, idx_hbm, out_hbm, idx_v, rows_v, semg, semo):
        wid = lax.axis_index("s") * nc + lax.axis_index("c")
        base = wid * b_per_w
        pltpu.sync_copy(idx_hbm.at[pl.ds(base, b_per_w)], idx_v)

        def fire(q, buf):
            return [pltpu.async_copy(
                table_hbm.at[idx_v.at[pl.ds(q * quarter + j * chunk, chunk)]],
                rows_v.at[buf, pl.ds(j * chunk, chunk)], semg)
                for j in range(quarter // chunk)]

        gh = fire(0, 0)
        oh = []
        for q in range(n_q):
            for c in gh:
                c.wait()
            oh.append(pltpu.async_copy(
                rows_v.at[q % 2],
                out_hbm.at[pl.ds(base + q * quarter, quarter)], semo))
            if q + 1 < n_q:
                if q >= 1:
                    oh[q - 1].wait()
                gh = fire(q + 1, (q + 1) % 2)
        oh[n_q - 2].wait()
        oh[n_q - 1].wait()

    return gather


def kernel(z, W):
    B, C, H, Wd = z.shape
    n = B * H * Wd
    n_steps = n // TN
    z_flat = jnp.transpose(z, (0, 2, 3, 1)).reshape(n, E_DIM)
    wt = W.T

    grid = (n_steps,)
    out_shapes = (
        jax.ShapeDtypeStruct((n, N_E), jnp.float32),
        jax.ShapeDtypeStruct((n_steps, 1, TN), jnp.int32),
        jax.ShapeDtypeStruct((1, 128), jnp.float32),
        jax.ShapeDtypeStruct((1, N_E), jnp.float32),
        jax.ShapeDtypeStruct((1, 128), jnp.float32),
    )
    in_specs = [
        pl.BlockSpec((TN, E_DIM), lambda i: (i, 0)),
        pl.BlockSpec((E_DIM, N_E), lambda i: (0, 0)),
    ]
    out_specs = (
        pl.BlockSpec((TN, N_E), lambda i: (i, 0)),
        pl.BlockSpec((1, 1, TN), lambda i: (i, 0, 0)),
        pl.BlockSpec((1, 128), lambda i: (0, 0)),
        pl.BlockSpec((1, N_E), lambda i: (0, 0)),
        pl.BlockSpec((1, 128), lambda i: (0, 0)),
    )
    one_hot, idx3, loss_o, _counts, perp_o = pl.pallas_call(
        functools.partial(_vq_kernel, n_total=n, n_steps=n_steps),
        grid=grid,
        in_specs=in_specs,
        out_specs=out_specs,
        out_shape=out_shapes,
        compiler_params=pltpu.CompilerParams(
            dimension_semantics=("arbitrary",)),
    )(z_flat, wt)

    indices = idx3.reshape(n)
    w_pad = jnp.pad(W, ((0, 0), (0, E_DIM)))
    zq_pad = _make_sc_gather(n)(w_pad, indices)
    zq_flat = zq_pad[:, :E_DIM]
    z_q = jnp.transpose(zq_flat.reshape(B, H, Wd, E_DIM), (0, 3, 1, 2))
    loss = loss_o[0, 0]
    perplexity = perp_o[0, 0]
    return (loss, z_q, perplexity, one_hot, indices)


# speedup vs baseline: 1.2071x; 1.0078x over previous
import functools

import jax
import jax.numpy as jnp
from jax import lax
from jax.experimental import pallas as pl
from jax.experimental.pallas import tpu as pltpu
from jax.experimental.pallas import tpu_sc as plsc

N_E = 1024
E_DIM = 64
BETA = 0.25
TN = 1024


def _vq_kernel(z_ref, wt_ref, oh_ref, idx_ref, loss_ref,
               counts_ref, perp_ref, *, n_total, n_steps):
    step = pl.program_id(0)

    z = z_ref[...]
    z2 = z + z
    wt = wt_ref[...]

    dot2 = jax.lax.dot_general(z2, wt, (((1,), (0,)), ((), ())),
                               preferred_element_type=jnp.float32)
    z_sq = 0.25 * jnp.sum(z2 * z2, axis=1, keepdims=True)
    e_sq = jnp.sum(wt * wt, axis=0, keepdims=True)
    d = (z_sq + e_sq) - dot2

    d_min = jnp.min(d, axis=1, keepdims=True)
    fiota = jax.lax.broadcasted_iota(jnp.int32, (TN, N_E), 1).astype(jnp.float32)
    idx_f = jnp.min(jnp.where(d == d_min, fiota, float(N_E)),
                    axis=1, keepdims=True)
    idx_ref[0] = jnp.transpose(idx_f.astype(jnp.int32))

    one_hot = (fiota == idx_f).astype(jnp.float32)
    oh_ref[...] = one_hot

    @pl.when(step == 0)
    def _init():
        loss_ref[...] = jnp.zeros_like(loss_ref)
        counts_ref[...] = jnp.zeros_like(counts_ref)
        perp_ref[...] = jnp.zeros_like(perp_ref)

    loss_ref[...] += jnp.full(loss_ref.shape, jnp.sum(d_min), jnp.float32)
    ones_row = jnp.ones((1, TN), jnp.float32)
    counts_ref[...] += jax.lax.dot_general(
        ones_row, one_hot, (((1,), (0,)), ((), ())),
        preferred_element_type=jnp.float32)

    @pl.when(step == n_steps - 1)
    def _finalize():
        loss_ref[...] = loss_ref[...] * (BETA / (n_total * E_DIM))
        p = counts_ref[...] / n_total
        ent = -jnp.sum(p * jnp.log(p + 1e-10))
        perp_ref[...] = jnp.full(perp_ref.shape, jnp.exp(ent), jnp.float32)


def _make_sc_gather(n):
    info = plsc.get_sparse_core_info()
    nc, ns = info.num_cores, info.num_subcores
    nw = nc * ns
    b_per_w = n // nw
    half = b_per_w // 2
    chunk = 128
    mesh = plsc.VectorSubcoreMesh(core_axis_name="c", subcore_axis_name="s")

    quarter = b_per_w // 4
    n_q = b_per_w // quarter

    @functools.partial(
        pl.kernel, mesh=mesh,
        out_type=jax.ShapeDtypeStruct((n, 2 * E_DIM), jnp.float32),
        scratch_types=[
            pltpu.VMEM((b_per_w,), jnp.int32),
            pltpu.VMEM((half, 2 * E_DIM), jnp.float32),
            pltpu.SemaphoreType.DMA,
        ],
    )
    def gather(table_hbm, idx_hbm, out_hbm, idx_v, rows_v, sem):
        wid = lax.axis_index("s") * nc + lax.axis_index("c")
        base = wid * b_per_w
        pltpu.sync_copy(idx_hbm.at[pl.ds(base, b_per_w)], idx_v)
        for h in range(2):
            copies = []
            for j in range(half // chunk):
                r = h * half + j * chunk
                copies.append(pltpu.async_copy(
                    table_hbm.at[idx_v.at[pl.ds(r, chunk)]],
                    rows_v.at[pl.ds(j * chunk, chunk)], sem))
            for c in copies:
                c.wait()
            pltpu.sync_copy(rows_v,
                            out_hbm.at[pl.ds(base + h * half, half)])

    return gather


def kernel(z, W):
    B, C, H, Wd = z.shape
    n = B * H * Wd
    n_steps = n // TN
    z_flat = jnp.transpose(z, (0, 2, 3, 1)).reshape(n, E_DIM)
    wt = W.T

    grid = (n_steps,)
    out_shapes = (
        jax.ShapeDtypeStruct((n, N_E), jnp.float32),
        jax.ShapeDtypeStruct((n_steps, 1, TN), jnp.int32),
        jax.ShapeDtypeStruct((1, 128), jnp.float32),
        jax.ShapeDtypeStruct((1, N_E), jnp.float32),
        jax.ShapeDtypeStruct((1, 128), jnp.float32),
    )
    in_specs = [
        pl.BlockSpec((TN, E_DIM), lambda i: (i, 0)),
        pl.BlockSpec((E_DIM, N_E), lambda i: (0, 0)),
    ]
    out_specs = (
        pl.BlockSpec((TN, N_E), lambda i: (i, 0)),
        pl.BlockSpec((1, 1, TN), lambda i: (i, 0, 0)),
        pl.BlockSpec((1, 128), lambda i: (0, 0)),
        pl.BlockSpec((1, N_E), lambda i: (0, 0)),
        pl.BlockSpec((1, 128), lambda i: (0, 0)),
    )
    one_hot, idx3, loss_o, _counts, perp_o = pl.pallas_call(
        functools.partial(_vq_kernel, n_total=n, n_steps=n_steps),
        grid=grid,
        in_specs=in_specs,
        out_specs=out_specs,
        out_shape=out_shapes,
        compiler_params=pltpu.CompilerParams(
            dimension_semantics=("arbitrary",)),
    )(z_flat, wt)

    indices = idx3.reshape(n)
    w_pad = jnp.pad(W, ((0, 0), (0, E_DIM)))
    zq_pad = _make_sc_gather(n)(w_pad, indices)
    zq_flat = zq_pad[:, :E_DIM]
    z_q = jnp.transpose(zq_flat.reshape(B, H, Wd, E_DIM), (0, 3, 1, 2))
    loss = loss_o[0, 0]
    perplexity = perp_o[0, 0]
    return (loss, z_q, perplexity, one_hot, indices)
